# baseline (speedup 1.0000x reference)
import functools

import jax
import jax.numpy as jnp
from jax import lax
from jax.experimental import pallas as pl
from jax.experimental.pallas import tpu as pltpu
from jax.experimental.pallas import tpu_sc as plsc

NC = 2
NS = 16
CHUNK = 128
PCH = 128
NBUF = 2


def _make_prop(n, d, n_chunks, n_acc, sizes, q):
    mesh = plsc.VectorSubcoreMesh(core_axis_name="c", subcore_axis_name="s",
                                  num_cores=NC, num_subcores=NS)
    half = n // NC

    @functools.partial(
        pl.kernel,
        out_type=jax.ShapeDtypeStruct((n, d // 128, 128), jnp.float32),
        mesh=mesh,
        scratch_types=[
            pltpu.VMEM((n_chunks // 2, PCH), jnp.int32),
            pltpu.VMEM((n_chunks // 2, PCH), jnp.int32),
            pltpu.VMEM((NBUF, PCH, d // 128, 128), jnp.float32),
            pltpu.VMEM_SHARED((n_acc, d // 128, 128), jnp.float32),
            [pltpu.SemaphoreType.DMA] * NBUF,
        ],
    )
    def prop(v_hbm, src_hbm, dst_hbm, ztile_hbm, out_hbm,
             src_v, dst_v, buf, acc, gsems):
        c = lax.axis_index("c")
        s = lax.axis_index("s")
        nh = n_chunks // 2
        n_iters = nh // NBUF

        def gather(j, b):
            pltpu.async_copy(
                v_hbm.at[plsc.Indices(src_v.at[j], ignored_value=-1)],
                buf.at[b], gsems[b])

        def drain_scatter(j, b):
            pltpu.make_async_copy(
                v_hbm.at[plsc.Indices(src_v.at[j], ignored_value=-1)],
                buf.at[b], gsems[b]).wait()
            pltpu.sync_copy(buf.at[b], acc.at[dst_v.at[j]], add=True)

        def body(k, carry):
            j0 = k * NBUF
            gather(j0 + NBUF - 1, NBUF - 1)
            drain_scatter(j0, 0)
            for b in range(NBUF - 1):
                @pl.when(k < n_iters - 1)
                def _(b=b):
                    gather(j0 + NBUF + b, b)
                drain_scatter(j0 + 1 + b, b + 1)
            return carry

        stripe = n_acc // NS
        for p in range(2):
            for h in range(2):
                pltpu.sync_copy(src_hbm.at[c, p, s, pl.ds(h * nh, nh)], src_v)
                pltpu.sync_copy(dst_hbm.at[c, p, s, pl.ds(h * nh, nh)], dst_v)
                for b in range(NBUF - 1):
                    gather(b, b)
                if h == 0:
                    pltpu.sync_copy(ztile_hbm, buf.at[NBUF - 1])
                    zrow = pl.multiple_of(s * stripe, 8)
                    for z0 in range(0, stripe, PCH):
                        zn = min(PCH, stripe - z0)
                        pltpu.sync_copy(
                            buf.at[NBUF - 1].at[pl.ds(0, zn)],
                            acc.at[pl.ds(zrow + z0, zn)])
                    plsc.subcore_barrier()
                lax.fori_loop(0, n_iters, body, 0)
            plsc.subcore_barrier()
            size = sizes[p]
            row0 = pl.multiple_of(s * stripe, 8)
            orow = pl.multiple_of(c * half + p * q + row0, 8)
            lastn = size - (NS - 1) * stripe

            @pl.when(s < NS - 1)
            def _(orow=orow, row0=row0):
                pltpu.sync_copy(acc.at[pl.ds(row0, stripe)],
                                out_hbm.at[pl.ds(orow, stripe)])

            @pl.when(s == NS - 1)
            def _(orow=orow, row0=row0, lastn=lastn):
                pltpu.sync_copy(acc.at[pl.ds(row0, lastn)],
                                out_hbm.at[pl.ds(orow, lastn)])

    return prop


def _make_hist(n, n_chunks, n_acc):
    mesh = plsc.VectorSubcoreMesh(core_axis_name="c", subcore_axis_name="s",
                                  num_cores=NC, num_subcores=NS)

    @functools.partial(
        pl.kernel,
        out_type=jax.ShapeDtypeStruct((2, n, 128), jnp.float32),
        mesh=mesh,
        scratch_types=[
            pltpu.VMEM((n_chunks, CHUNK), jnp.int32),
            pltpu.VMEM((CHUNK, 128), jnp.float32),
            pltpu.VMEM_SHARED((n_acc, 128), jnp.float32),
            pltpu.SemaphoreType.DMA,
        ],
    )
    def hist(src_hbm, ones_hbm, ztile_hbm, out_hbm, src_v, ones_v, acc, sem):
        c = lax.axis_index("c")
        s = lax.axis_index("s")
        pltpu.sync_copy(src_hbm.at[c, s], src_v)
        zrow = pl.multiple_of(s * (n_acc // NS), 8)
        pltpu.sync_copy(ztile_hbm, ones_v)
        for k in range(n_acc // NS // CHUNK):
            pltpu.sync_copy(ones_v, acc.at[pl.ds(zrow + k * CHUNK, CHUNK)])
        pltpu.sync_copy(ones_hbm, ones_v)
        plsc.subcore_barrier()

        def body(j, carry):
            pltpu.sync_copy(ones_v, acc.at[src_v.at[j]], add=True)
            return carry

        lax.fori_loop(0, n_chunks, body, 0)
        plsc.subcore_barrier()
        stripe = n_acc // NS
        last = n - (NS - 1) * stripe
        row0 = pl.multiple_of(s * stripe, 8)

        @pl.when(s < NS - 1)
        def _():
            pltpu.sync_copy(acc.at[pl.ds(row0, stripe)],
                            out_hbm.at[c].at[pl.ds(row0, stripe)])

        @pl.when(s == NS - 1)
        def _():
            pltpu.sync_copy(acc.at[pl.ds(row0, last)],
                            out_hbm.at[c].at[pl.ds(row0, last)])

    return hist


def _k1_body(hist_ref, x_ref, dinv_ref, xs_ref):
    deg = hist_ref[0, :, 0] + hist_ref[1, :, 0]
    dinv = jnp.where(deg > 0, lax.rsqrt(deg), 0.0)[:, None]
    dinv_ref[...] = dinv
    xs_ref[...] = dinv * x_ref[...]


def _k2_body(g1_ref, dinv_ref, out_ref):
    dv = dinv_ref[...]
    out_ref[...] = (dv * dv) * g1_ref[...]


def _k3_body(x_ref, g1_ref, g2_ref, dinv_ref, w1_ref, b1_ref, w2_ref,
             u0m2_ref, u1d_ref, y3_ref):
    dv = dinv_ref[...]
    t1 = -dv * g1_ref[...]
    t2 = (2.0 * dv) * g2_ref[...]
    h = (x_ref[...] @ (w1_ref[0] - w1_ref[2])
         + t1 @ w1_ref[1]
         + t2 @ w1_ref[2]
         + b1_ref[...])
    h = jnp.maximum(h, 0.0)
    u0m2_ref[...] = h @ (w2_ref[0] - w2_ref[2])
    u1d_ref[...] = dv * (h @ w2_ref[1])
    y3_ref[...] = dv * (h @ w2_ref[2])


def _k5_body(u1d_ref, g3_ref, dinv_ref, z_ref):
    dv = dinv_ref[...]
    z_ref[...] = u1d_ref[...] - (2.0 * dv * dv) * g3_ref[...]


def _k6_body(u_ref, g4_ref, dinv_ref, b2_ref, out_ref):
    dv = dinv_ref[...]
    o = u_ref[...] - dv * g4_ref[...] + b2_ref[...]
    m = jnp.max(o, axis=1, keepdims=True)
    lse = jnp.log(jnp.sum(jnp.exp(o - m), axis=1, keepdims=True))
    out_ref[...] = o - m - lse


def kernel(x, edge_index, W1, b1, W2, b2):
    n, din = x.shape
    e = edge_index.shape[1]
    dhid = W1.shape[2]
    dout = W2.shape[2]

    epad = NC * NS * CHUNK * (-(-e // (NC * NS * CHUNK)))
    nchp = epad // (NS * PCH)
    nch_h = epad // (NC * NS * CHUNK)
    pad = epad - e
    src = edge_index[0]
    dst = edge_index[1]
    src_p = jnp.concatenate([src, jnp.full((pad,), -1, jnp.int32)])
    dst_p = jnp.concatenate([dst, jnp.full((pad,), -1, jnp.int32)])

    half = n // NC
    q = 8 * (-(-(half // 2) // 8))
    n_acc = NS * 8 * (-(-(q + 8) // (NS * 8)))
    dump = n_acc - 8
    sizes = (q, half - q)
    lo = [0, q, half, half + q]
    hi = [q, half, half + q, n]
    srcs, dsts = [], []
    for r in range(4):
        in_r = (dst_p >= lo[r]) & (dst_p < hi[r])
        srcs.append(jnp.where(in_r, src_p, -1).reshape(NS, nchp, PCH))
        dsts.append(jnp.where(in_r, dst_p - lo[r], dump).reshape(NS, nchp, PCH))
    src_prop = jnp.stack(srcs).reshape(NC, 2, NS, nchp, PCH)
    dst_prop = jnp.stack(dsts).reshape(NC, 2, NS, nchp, PCH)
    src_hist = jnp.concatenate(
        [src, jnp.full((pad,), n, jnp.int32)]).reshape(NC, NS, nch_h, CHUNK)

    n_acc_h = NS * CHUNK * (-(-n // (NS * CHUNK)))
    ztile = jnp.zeros((CHUNK, 128), jnp.float32)
    zprop = jnp.zeros((PCH, din // 128, 128), jnp.float32)
    ones_tile = jnp.ones((CHUNK, 128), jnp.float32)

    prop = _make_prop(n, din, nchp, n_acc, sizes, q)
    hist = _make_hist(n, nch_h, n_acc_h)

    B = 1000
    grid = (n // B,)
    f32 = jnp.float32

    spec_rows = lambda w: pl.BlockSpec((B, w), lambda i: (i, 0))
    spec_dinv = pl.BlockSpec((B, 1), lambda i: (i, 0))
    spec_full = lambda shp: pl.BlockSpec(shp, lambda i: (0,) * len(shp))

    k1 = pl.pallas_call(
        _k1_body,
        grid=grid,
        in_specs=[pl.BlockSpec((2, B, 128), lambda i: (0, i, 0)),
                  spec_rows(din)],
        out_specs=[spec_dinv, spec_rows(din)],
        out_shape=[jax.ShapeDtypeStruct((n, 1), f32),
                   jax.ShapeDtypeStruct((n, din), f32)],
    )
    k2 = pl.pallas_call(
        _k2_body,
        grid=grid,
        in_specs=[spec_rows(din), spec_dinv],
        out_specs=spec_rows(din),
        out_shape=jax.ShapeDtypeStruct((n, din), f32),
    )
    k3 = pl.pallas_call(
        _k3_body,
        grid=grid,
        in_specs=[spec_rows(din), spec_rows(din), spec_rows(din), spec_dinv,
                  spec_full((3, din, dhid)), spec_full((1, dhid)),
                  spec_full((3, dhid, dout))],
        out_specs=[spec_rows(dout), spec_rows(dout), spec_rows(dout)],
        out_shape=[jax.ShapeDtypeStruct((n, dout), f32),
                   jax.ShapeDtypeStruct((n, dout), f32),
                   jax.ShapeDtypeStruct((n, dout), f32)],
    )
    k5 = pl.pallas_call(
        _k5_body,
        grid=grid,
        in_specs=[spec_rows(dout), spec_rows(dout), spec_dinv],
        out_specs=spec_rows(dout),
        out_shape=jax.ShapeDtypeStruct((n, dout), f32),
    )
    k6 = pl.pallas_call(
        _k6_body,
        grid=grid,
        in_specs=[spec_rows(dout), spec_rows(dout), spec_dinv,
                  spec_full((1, dout))],
        out_specs=spec_rows(dout),
        out_shape=jax.ShapeDtypeStruct((n, dout), f32),
    )

    hist_out = hist(src_hist, ones_tile, ztile)
    dinv, xs = k1(hist_out, x)
    g1 = prop(xs.reshape(n, din // 128, 128), src_prop, dst_prop,
              zprop).reshape(n, din)
    c2 = k2(g1, dinv)
    g2 = prop(c2.reshape(n, din // 128, 128), src_prop, dst_prop,
              zprop).reshape(n, din)
    u0m2, u1d, y3 = k3(x, g1, g2, dinv, W1, b1.reshape(1, dhid), W2)
    g3 = prop(y3.reshape(n, dout // 128, 128), src_prop, dst_prop,
              zprop).reshape(n, dout)
    z = k5(u1d, g3, dinv)
    g4 = prop(z.reshape(n, dout // 128, 128), src_prop, dst_prop,
              zprop).reshape(n, dout)
    return k6(u0m2, g4, dinv, b2.reshape(1, dout))


# speedup vs baseline: 1.2313x; 1.2313x over previous
import functools

import jax
import jax.numpy as jnp
from jax import lax
from jax.experimental import pallas as pl
from jax.experimental.pallas import tpu as pltpu
from jax.experimental.pallas import tpu_sc as plsc

NC = 2
NS = 16
CHUNK = 128


PCH = 128
NBUF = 2


def _make_prop(n, d_half, n_chunks, n_acc, rows_out):
    mesh = plsc.VectorSubcoreMesh(core_axis_name="c", subcore_axis_name="s", num_cores=NC, num_subcores=NS)

    @functools.partial(
        pl.kernel,
        out_type=jax.ShapeDtypeStruct((2 * n, d_half), jnp.float32),
        mesh=mesh,
        scratch_types=[
            pltpu.VMEM((n_chunks // 2, PCH), jnp.int32),
            pltpu.VMEM((n_chunks // 2, PCH), jnp.int32),
            pltpu.VMEM((NBUF, PCH, d_half), jnp.float32),
            pltpu.VMEM_SHARED((n_acc, d_half), jnp.float32),
            [pltpu.SemaphoreType.DMA] * NBUF,
        ],
    )
    def prop(v_hbm, src_hbm, dst_hbm, ztile_hbm, out_hbm,
             src_v, dst_v, buf, acc, gsems):
        c = lax.axis_index("c")
        s = lax.axis_index("s")
        nh = n_chunks // 2
        n_iters = nh // NBUF

        def gather(j, b):
            pltpu.async_copy(v_hbm.at[src_v.at[j]], buf.at[b], gsems[b])

        def drain_scatter(j, b):
            pltpu.make_async_copy(v_hbm.at[src_v.at[j]], buf.at[b],
                                  gsems[b]).wait()
            pltpu.sync_copy(buf.at[b], acc.at[dst_v.at[j]], add=True)

        def body(k, carry):
            j0 = k * NBUF
            gather(j0 + NBUF - 1, NBUF - 1)
            drain_scatter(j0, 0)
            for b in range(NBUF - 1):
                @pl.when(k < n_iters - 1)
                def _(b=b):
                    gather(j0 + NBUF + b, b)
                drain_scatter(j0 + 1 + b, b + 1)
            return carry

        for h in range(2):
            pltpu.sync_copy(src_hbm.at[c, s, pl.ds(h * nh, nh)], src_v)
            pltpu.sync_copy(dst_hbm.at[s, pl.ds(h * nh, nh)], dst_v)
            if h == 0:
                for b in range(NBUF - 1):
                    gather(b, b)
                pltpu.sync_copy(ztile_hbm.at[pl.ds(0, PCH)], buf.at[1])
                zrow = pl.multiple_of(s * (n_acc // NS), 8)
                for k in range(n_acc // NS // PCH):
                    pltpu.sync_copy(buf.at[1],
                                    acc.at[pl.ds(zrow + k * PCH, PCH)])
                plsc.subcore_barrier()
            else:
                for b in range(NBUF - 1):
                    gather(b, b)
            lax.fori_loop(0, n_iters, body, 0)
        plsc.subcore_barrier()
        stripe = n_acc // NS
        last = n - (NS - 1) * stripe
        row0 = pl.multiple_of(s * stripe, 8)
        orow0 = pl.multiple_of(c * n + row0, 8)

        @pl.when(s < NS - 1)
        def _():
            pltpu.sync_copy(acc.at[pl.ds(row0, stripe)],
                            out_hbm.at[pl.ds(orow0, stripe)])

        @pl.when(s == NS - 1)
        def _():
            pltpu.sync_copy(acc.at[pl.ds(row0, last)],
                            out_hbm.at[pl.ds(orow0, last)])

    return prop


def _make_hist(n, n_chunks, n_acc):
    mesh = plsc.VectorSubcoreMesh(core_axis_name="c", subcore_axis_name="s", num_cores=NC, num_subcores=NS)

    @functools.partial(
        pl.kernel,
        out_type=jax.ShapeDtypeStruct((2, n, 128), jnp.float32),
        mesh=mesh,
        scratch_types=[
            pltpu.VMEM((n_chunks, CHUNK), jnp.int32),
            pltpu.VMEM((CHUNK, 128), jnp.float32),
            pltpu.VMEM_SHARED((n_acc, 128), jnp.float32),
            pltpu.SemaphoreType.DMA,
        ],
    )
    def hist(src_hbm, ones_hbm, ztile_hbm, out_hbm, src_v, ones_v, acc, sem):
        c = lax.axis_index("c")
        s = lax.axis_index("s")
        pltpu.sync_copy(src_hbm.at[c, s], src_v)
        zrow = pl.multiple_of(s * (n_acc // NS), 8)
        pltpu.sync_copy(ztile_hbm, ones_v)
        for k in range(n_acc // NS // CHUNK):
            pltpu.sync_copy(ones_v, acc.at[pl.ds(zrow + k * CHUNK, CHUNK)])
        pltpu.sync_copy(ones_hbm, ones_v)
        plsc.subcore_barrier()

        def body(j, carry):
            pltpu.sync_copy(ones_v, acc.at[src_v.at[j]], add=True)
            return carry

        lax.fori_loop(0, n_chunks, body, 0)
        plsc.subcore_barrier()
        stripe = n_acc // NS
        last = n - (NS - 1) * stripe
        row0 = pl.multiple_of(s * stripe, 8)

        @pl.when(s < NS - 1)
        def _():
            pltpu.sync_copy(acc.at[pl.ds(row0, stripe)],
                            out_hbm.at[c].at[pl.ds(row0, stripe)])

        @pl.when(s == NS - 1)
        def _():
            pltpu.sync_copy(acc.at[pl.ds(row0, last)],
                            out_hbm.at[c].at[pl.ds(row0, last)])

    return hist


def _k1_body(hist_ref, x_ref, dinv_ref, xs_ref):
    deg = hist_ref[0, :, 0] + hist_ref[1, :, 0]
    dinv = jnp.where(deg > 0, lax.rsqrt(deg), 0.0)[:, None]
    dinv_ref[...] = dinv
    x = x_ref[...]
    xs_ref[0] = dinv * x[:, :128]
    xs_ref[1] = dinv * x[:, 128:]


def _k2_body(g1_ref, dinv_ref, out_ref):
    dv = dinv_ref[...]
    dv2 = dv * dv
    out_ref[0] = dv2 * g1_ref[0]
    out_ref[1] = dv2 * g1_ref[1]


def _k3a_body(x_ref, g1_ref, g2_ref, dinv_ref, w1_ref, b1_ref, w2_ref,
              h_ref, y3_ref):
    dv = dinv_ref[...]
    g1 = jnp.concatenate([g1_ref[0], g1_ref[1]], axis=1)
    g2 = jnp.concatenate([g2_ref[0], g2_ref[1]], axis=1)
    t1 = -dv * g1
    t2 = (2.0 * dv) * g2
    h = (x_ref[...] @ (w1_ref[0] - w1_ref[2])
         + t1 @ w1_ref[1]
         + t2 @ w1_ref[2]
         + b1_ref[...])
    h = jnp.maximum(h, 0.0)
    h_ref[...] = h
    y3 = dv * (h @ w2_ref[2])
    y3_ref[0] = y3[:, :128]
    y3_ref[1] = y3[:, 128:]


def _k3b_body(h_ref, dinv_ref, w2_ref, u0m2_ref, u1d_ref):
    dv = dinv_ref[...]
    h = h_ref[...]
    u0m2_ref[...] = h @ (w2_ref[0] - w2_ref[2])
    u1 = dv * (h @ w2_ref[1])
    u1d_ref[0] = u1[:, :128]
    u1d_ref[1] = u1[:, 128:]


def _k5_body(u1d_ref, g3_ref, dinv_ref, z_ref):
    dv = dinv_ref[...]
    dv2 = 2.0 * dv * dv
    z_ref[0] = u1d_ref[0] - dv2 * g3_ref[0]
    z_ref[1] = u1d_ref[1] - dv2 * g3_ref[1]


def _k6_body(u_ref, g4_ref, dinv_ref, b2_ref, out_ref):
    dv = dinv_ref[...]
    g4 = jnp.concatenate([g4_ref[0], g4_ref[1]], axis=1)
    o = u_ref[...] - dv * g4 + b2_ref[...]
    m = jnp.max(o, axis=1, keepdims=True)
    lse = jnp.log(jnp.sum(jnp.exp(o - m), axis=1, keepdims=True))
    out_ref[...] = o - m - lse


def kernel(x, edge_index, W1, b1, W2, b2):
    n, din = x.shape
    e = edge_index.shape[1]
    dhid = W1.shape[2]
    dout = W2.shape[2]
    dh = din // 2

    epad = NC * NS * CHUNK * (-(-e // (NC * NS * CHUNK)))
    nch = epad // (NS * CHUNK)
    nch_h = epad // (NC * NS * CHUNK)
    pad = epad - e
    src = edge_index[0]
    dst = edge_index[1]
    src_p = jnp.concatenate([src, jnp.zeros((pad,), jnp.int32)])
    dst_p = jnp.concatenate([dst, jnp.full((pad,), n, jnp.int32)])
    nchp = epad // (NS * PCH)
    src_r = src_p.reshape(NS, nchp, PCH)
    src_prop = jnp.stack([src_r, src_r + n])
    dst_prop = dst_p.reshape(NS, nchp, PCH)
    src_hist = jnp.concatenate(
        [src, jnp.full((pad,), n, jnp.int32)]).reshape(NC, NS, nch_h, CHUNK)

    n_acc = NS * CHUNK * (-(-n // (NS * CHUNK)))
    rows_out = n // NS
    ztile = jnp.zeros((CHUNK, dh), jnp.float32)
    ones_tile = jnp.ones((CHUNK, 128), jnp.float32)

    prop = _make_prop(n, dh, nchp, n_acc, rows_out)
    hist = _make_hist(n, nch_h, n_acc)

    B = 1000
    grid = (n // B,)
    f32 = jnp.float32

    spec_split = pl.BlockSpec((2, B, dh), lambda i: (0, i, 0))
    spec_rows = lambda w: pl.BlockSpec((B, w), lambda i: (i, 0))
    spec_dinv = pl.BlockSpec((B, 1), lambda i: (i, 0))
    spec_full = lambda shp: pl.BlockSpec(shp, lambda i: (0,) * len(shp))

    k1 = pl.pallas_call(
        _k1_body,
        grid=grid,
        in_specs=[pl.BlockSpec((2, B, 128), lambda i: (0, i, 0)), spec_rows(din)],
        out_specs=[spec_dinv, spec_split],
        out_shape=[jax.ShapeDtypeStruct((n, 1), f32),
                   jax.ShapeDtypeStruct((2, n, dh), f32)],
    )
    k2 = pl.pallas_call(
        _k2_body,
        grid=grid,
        in_specs=[spec_split, spec_dinv],
        out_specs=spec_split,
        out_shape=jax.ShapeDtypeStruct((2, n, dh), f32),
    )
    k3a = pl.pallas_call(
        _k3a_body,
        grid=grid,
        in_specs=[spec_rows(din), spec_split, spec_split, spec_dinv,
                  spec_full((3, din, dhid)), spec_full((1, dhid)),
                  spec_full((3, dhid, dout))],
        out_specs=[spec_rows(dhid), spec_split],
        out_shape=[jax.ShapeDtypeStruct((n, dhid), f32),
                   jax.ShapeDtypeStruct((2, n, dh), f32)],
    )
    k3b = pl.pallas_call(
        _k3b_body,
        grid=grid,
        in_specs=[spec_rows(dhid), spec_dinv, spec_full((3, dhid, dout))],
        out_specs=[spec_rows(dout), spec_split],
        out_shape=[jax.ShapeDtypeStruct((n, dout), f32),
                   jax.ShapeDtypeStruct((2, n, dh), f32)],
    )
    k5 = pl.pallas_call(
        _k5_body,
        grid=grid,
        in_specs=[spec_split, spec_split, spec_dinv],
        out_specs=spec_split,
        out_shape=jax.ShapeDtypeStruct((2, n, dh), f32),
    )
    k6 = pl.pallas_call(
        _k6_body,
        grid=grid,
        in_specs=[spec_rows(dout), spec_split, spec_dinv, spec_full((1, dout))],
        out_specs=spec_rows(dout),
        out_shape=jax.ShapeDtypeStruct((n, dout), f32),
    )

    hist_out = hist(src_hist, ones_tile, ztile)
    dinv, xs = k1(hist_out, x)
    g1 = prop(xs.reshape(2 * n, dh), src_prop, dst_prop, ztile)
    g1 = g1.reshape(2, n, dh)
    c2 = k2(g1, dinv)
    g2 = prop(c2.reshape(2 * n, dh), src_prop, dst_prop, ztile)
    g2 = g2.reshape(2, n, dh)
    hmat, y3 = k3a(x, g1, g2, dinv, W1, b1.reshape(1, dhid), W2)
    g3 = prop(y3.reshape(2 * n, dh), src_prop, dst_prop, ztile)
    u0m2, u1d = k3b(hmat, dinv, W2)
    g3 = g3.reshape(2, n, dh)
    z = k5(u1d, g3, dinv)
    g4 = prop(z.reshape(2 * n, dh), src_prop, dst_prop, ztile)
    g4 = g4.reshape(2, n, dh)
    return k6(u0m2, g4, dinv, b2.reshape(1, dout))


# speedup vs baseline: 1.2940x; 1.0509x over previous
import functools

import jax
import jax.numpy as jnp
from jax import lax
from jax.experimental import pallas as pl
from jax.experimental.pallas import tpu as pltpu
from jax.experimental.pallas import tpu_sc as plsc

NC = 2
NS = 16
CHUNK = 128


PCH = 128
NBUF = 2


def _make_prop(n, d_half, n_chunks, n_acc, rows_out):
    mesh = plsc.VectorSubcoreMesh(core_axis_name="c", subcore_axis_name="s", num_cores=NC, num_subcores=NS)

    @functools.partial(
        pl.kernel,
        out_type=jax.ShapeDtypeStruct((2 * n, d_half), jnp.float32),
        mesh=mesh,
        scratch_types=[
            pltpu.VMEM((n_chunks // 2, PCH), jnp.int32),
            pltpu.VMEM((n_chunks // 2, PCH), jnp.int32),
            pltpu.VMEM((NBUF, PCH, d_half), jnp.float32),
            pltpu.VMEM_SHARED((n_acc, d_half), jnp.float32),
            [pltpu.SemaphoreType.DMA] * NBUF,
        ],
    )
    def prop(v_hbm, src_hbm, dst_hbm, ztile_hbm, out_hbm,
             src_v, dst_v, buf, acc, gsems):
        c = lax.axis_index("c")
        s = lax.axis_index("s")
        nh = n_chunks // 2
        n_iters = nh // NBUF

        def gather(j, b):
            pltpu.async_copy(v_hbm.at[src_v.at[j]], buf.at[b], gsems[b])

        def drain_scatter(j, b):
            pltpu.make_async_copy(v_hbm.at[src_v.at[j]], buf.at[b],
                                  gsems[b]).wait()
            pltpu.sync_copy(buf.at[b], acc.at[dst_v.at[j]], add=True)

        def body(k, carry):
            j0 = k * NBUF
            gather(j0 + NBUF - 1, NBUF - 1)
            drain_scatter(j0, 0)
            for b in range(NBUF - 1):
                @pl.when(k < n_iters - 1)
                def _(b=b):
                    gather(j0 + NBUF + b, b)
                drain_scatter(j0 + 1 + b, b + 1)
            return carry

        for h in range(2):
            pltpu.sync_copy(src_hbm.at[c, s, pl.ds(h * nh, nh)], src_v)
            pltpu.sync_copy(dst_hbm.at[s, pl.ds(h * nh, nh)], dst_v)
            if h == 0:
                for b in range(NBUF - 1):
                    gather(b, b)
                pltpu.sync_copy(ztile_hbm.at[pl.ds(0, PCH)], buf.at[1])
                zrow = pl.multiple_of(s * (n_acc // NS), 8)
                for k in range(n_acc // NS // PCH):
                    pltpu.sync_copy(buf.at[1],
                                    acc.at[pl.ds(zrow + k * PCH, PCH)])
                plsc.subcore_barrier()
            else:
                for b in range(NBUF - 1):
                    gather(b, b)
            lax.fori_loop(0, n_iters, body, 0)
        plsc.subcore_barrier()
        stripe = n_acc // NS
        last = n - (NS - 1) * stripe
        row0 = pl.multiple_of(s * stripe, 8)
        orow0 = pl.multiple_of(c * n + row0, 8)

        @pl.when(s < NS - 1)
        def _():
            pltpu.sync_copy(acc.at[pl.ds(row0, stripe)],
                            out_hbm.at[pl.ds(orow0, stripe)])

        @pl.when(s == NS - 1)
        def _():
            pltpu.sync_copy(acc.at[pl.ds(row0, last)],
                            out_hbm.at[pl.ds(orow0, last)])

    return prop


def _make_hist(n, n_chunks, n_acc):
    mesh = plsc.VectorSubcoreMesh(core_axis_name="c", subcore_axis_name="s", num_cores=NC, num_subcores=NS)

    @functools.partial(
        pl.kernel,
        out_type=jax.ShapeDtypeStruct((2, n, 128), jnp.float32),
        mesh=mesh,
        scratch_types=[
            pltpu.VMEM((n_chunks, CHUNK), jnp.int32),
            pltpu.VMEM((CHUNK, 128), jnp.float32),
            pltpu.VMEM_SHARED((n_acc, 128), jnp.float32),
            pltpu.SemaphoreType.DMA,
        ],
    )
    def hist(src_hbm, ones_hbm, ztile_hbm, out_hbm, src_v, ones_v, acc, sem):
        c = lax.axis_index("c")
        s = lax.axis_index("s")
        pltpu.sync_copy(src_hbm.at[c, s], src_v)
        zrow = pl.multiple_of(s * (n_acc // NS), 8)
        pltpu.sync_copy(ztile_hbm, ones_v)
        for k in range(n_acc // NS // CHUNK):
            pltpu.sync_copy(ones_v, acc.at[pl.ds(zrow + k * CHUNK, CHUNK)])
        pltpu.sync_copy(ones_hbm, ones_v)
        plsc.subcore_barrier()

        def body(j, carry):
            pltpu.sync_copy(ones_v, acc.at[src_v.at[j]], add=True)
            return carry

        lax.fori_loop(0, n_chunks, body, 0)
        plsc.subcore_barrier()
        stripe = n_acc // NS
        last = n - (NS - 1) * stripe
        row0 = pl.multiple_of(s * stripe, 8)

        @pl.when(s < NS - 1)
        def _():
            pltpu.sync_copy(acc.at[pl.ds(row0, stripe)],
                            out_hbm.at[c].at[pl.ds(row0, stripe)])

        @pl.when(s == NS - 1)
        def _():
            pltpu.sync_copy(acc.at[pl.ds(row0, last)],
                            out_hbm.at[c].at[pl.ds(row0, last)])

    return hist


def _k1_body(hist_ref, x_ref, dinv_ref, xs_ref):
    deg = hist_ref[0, :, 0] + hist_ref[1, :, 0]
    dinv = jnp.where(deg > 0, lax.rsqrt(deg), 0.0)[:, None]
    dinv_ref[...] = dinv
    x = x_ref[...]
    xs_ref[0] = dinv * x[:, :128]
    xs_ref[1] = dinv * x[:, 128:]


def _k2_body(g1_ref, dinv_ref, out_ref):
    dv = dinv_ref[...]
    dv2 = dv * dv
    out_ref[0] = dv2 * g1_ref[0]
    out_ref[1] = dv2 * g1_ref[1]


def _k3_body(x_ref, g1_ref, g2_ref, dinv_ref, w1_ref, b1_ref, w2_ref,
             u0m2_ref, u1d_ref, y3_ref):
    dv = dinv_ref[...]
    g1 = jnp.concatenate([g1_ref[0], g1_ref[1]], axis=1)
    g2 = jnp.concatenate([g2_ref[0], g2_ref[1]], axis=1)
    t1 = -dv * g1
    t2 = (2.0 * dv) * g2
    h = (x_ref[...] @ (w1_ref[0] - w1_ref[2])
         + t1 @ w1_ref[1]
         + t2 @ w1_ref[2]
         + b1_ref[...])
    h = jnp.maximum(h, 0.0)
    u0m2_ref[...] = h @ (w2_ref[0] - w2_ref[2])
    u1 = h @ w2_ref[1]
    u2 = h @ w2_ref[2]
    u1d_ref[0] = dv * u1[:, :128]
    u1d_ref[1] = dv * u1[:, 128:]
    y3_ref[0] = dv * u2[:, :128]
    y3_ref[1] = dv * u2[:, 128:]


def _k5_body(u1d_ref, g3_ref, dinv_ref, z_ref):
    dv = dinv_ref[...]
    dv2 = 2.0 * dv * dv
    z_ref[0] = u1d_ref[0] - dv2 * g3_ref[0]
    z_ref[1] = u1d_ref[1] - dv2 * g3_ref[1]


def _k6_body(u_ref, g4_ref, dinv_ref, b2_ref, out_ref):
    dv = dinv_ref[...]
    g4 = jnp.concatenate([g4_ref[0], g4_ref[1]], axis=1)
    o = u_ref[...] - dv * g4 + b2_ref[...]
    m = jnp.max(o, axis=1, keepdims=True)
    lse = jnp.log(jnp.sum(jnp.exp(o - m), axis=1, keepdims=True))
    out_ref[...] = o - m - lse


def kernel(x, edge_index, W1, b1, W2, b2):
    n, din = x.shape
    e = edge_index.shape[1]
    dhid = W1.shape[2]
    dout = W2.shape[2]
    dh = din // 2

    epad = NC * NS * CHUNK * (-(-e // (NC * NS * CHUNK)))
    nch = epad // (NS * CHUNK)
    nch_h = epad // (NC * NS * CHUNK)
    pad = epad - e
    src = edge_index[0]
    dst = edge_index[1]
    src_p = jnp.concatenate([src, jnp.zeros((pad,), jnp.int32)])
    dst_p = jnp.concatenate([dst, jnp.full((pad,), n, jnp.int32)])
    nchp = epad // (NS * PCH)
    src_r = src_p.reshape(NS, nchp, PCH)
    src_prop = jnp.stack([src_r, src_r + n])
    dst_prop = dst_p.reshape(NS, nchp, PCH)
    src_hist = jnp.concatenate(
        [src, jnp.full((pad,), n, jnp.int32)]).reshape(NC, NS, nch_h, CHUNK)

    n_acc = NS * CHUNK * (-(-n // (NS * CHUNK)))
    rows_out = n // NS
    ztile = jnp.zeros((CHUNK, dh), jnp.float32)
    ones_tile = jnp.ones((CHUNK, 128), jnp.float32)

    prop = _make_prop(n, dh, nchp, n_acc, rows_out)
    hist = _make_hist(n, nch_h, n_acc)

    B = 1000
    grid = (n // B,)
    f32 = jnp.float32

    spec_split = pl.BlockSpec((2, B, dh), lambda i: (0, i, 0))
    spec_rows = lambda w: pl.BlockSpec((B, w), lambda i: (i, 0))
    spec_dinv = pl.BlockSpec((B, 1), lambda i: (i, 0))
    spec_full = lambda shp: pl.BlockSpec(shp, lambda i: (0,) * len(shp))

    k1 = pl.pallas_call(
        _k1_body,
        grid=grid,
        in_specs=[pl.BlockSpec((2, B, 128), lambda i: (0, i, 0)), spec_rows(din)],
        out_specs=[spec_dinv, spec_split],
        out_shape=[jax.ShapeDtypeStruct((n, 1), f32),
                   jax.ShapeDtypeStruct((2, n, dh), f32)],
    )
    k2 = pl.pallas_call(
        _k2_body,
        grid=grid,
        in_specs=[spec_split, spec_dinv],
        out_specs=spec_split,
        out_shape=jax.ShapeDtypeStruct((2, n, dh), f32),
    )
    k3 = pl.pallas_call(
        _k3_body,
        grid=grid,
        in_specs=[spec_rows(din), spec_split, spec_split, spec_dinv,
                  spec_full((3, din, dhid)), spec_full((1, dhid)),
                  spec_full((3, dhid, dout))],
        out_specs=[spec_rows(dout), spec_split, spec_split],
        out_shape=[jax.ShapeDtypeStruct((n, dout), f32),
                   jax.ShapeDtypeStruct((2, n, dh), f32),
                   jax.ShapeDtypeStruct((2, n, dh), f32)],
    )
    k5 = pl.pallas_call(
        _k5_body,
        grid=grid,
        in_specs=[spec_split, spec_split, spec_dinv],
        out_specs=spec_split,
        out_shape=jax.ShapeDtypeStruct((2, n, dh), f32),
    )
    k6 = pl.pallas_call(
        _k6_body,
        grid=grid,
        in_specs=[spec_rows(dout), spec_split, spec_dinv, spec_full((1, dout))],
        out_specs=spec_rows(dout),
        out_shape=jax.ShapeDtypeStruct((n, dout), f32),
    )

    hist_out = hist(src_hist, ones_tile, ztile)
    dinv, xs = k1(hist_out, x)
    g1 = prop(xs.reshape(2 * n, dh), src_prop, dst_prop, ztile)
    g1 = g1.reshape(2, n, dh)
    c2 = k2(g1, dinv)
    g2 = prop(c2.reshape(2 * n, dh), src_prop, dst_prop, ztile)
    g2 = g2.reshape(2, n, dh)
    u0m2, u1d, y3 = k3(x, g1, g2, dinv, W1, b1.reshape(1, dhid), W2)
    g3 = prop(y3.reshape(2 * n, dh), src_prop, dst_prop, ztile)
    g3 = g3.reshape(2, n, dh)
    z = k5(u1d, g3, dinv)
    g4 = prop(z.reshape(2 * n, dh), src_prop, dst_prop, ztile)
    g4 = g4.reshape(2, n, dh)
    return k6(u0m2, g4, dinv, b2.reshape(1, dout))
